# Initial kernel scaffold; baseline (speedup 1.0000x reference)
import jax
import jax.numpy as jnp
from jax.experimental import pallas as pl


def kernel(x0, x1, x2, x3, x4, w, b):
    raise NotImplementedError("write your pallas kernel here")


# speedup vs baseline: 2.2300x; 2.2300x over previous
import jax
import jax.numpy as jnp
from jax.experimental import pallas as pl
from jax.experimental.pallas import tpu as pltpu


def _residual_kernel(x0_ref, x1_ref, x2_ref, x3_ref, x4_ref, w_ref, out_ref):
    hf = w_ref.shape[1]
    x0 = x0_ref[...]
    w = w_ref[...]
    for h, x_ref in enumerate((x1_ref, x2_ref, x3_ref, x4_ref)):
        out_ref[:, h * hf:(h + 1) * hf] = jnp.dot(
            x_ref[...] - x0, w, preferred_element_type=jnp.float32
        ).astype(out_ref.dtype)


def _pick_tile(n, target):
    target = max(1, min(target, n))
    for t in range(target, 0, -1):
        if n % t == 0 and (t % 8 == 0 or t == n):
            return t
    return n


def kernel(x0, x1, x2, x3, x4, w, b):
    del b
    n, fin = x0.shape
    hf = w.shape[1]
    out_cols = 4 * hf

    tile = _pick_tile(n, 1024)
    n_tiles = n // tile

    x_spec = pl.BlockSpec((tile, fin), lambda i: (i, 0))
    out = pl.pallas_call(
        _residual_kernel,
        out_shape=jax.ShapeDtypeStruct((n, out_cols), x0.dtype),
        grid=(n_tiles,),
        in_specs=[x_spec] * 5 + [pl.BlockSpec((fin, hf), lambda i: (0, 0))],
        out_specs=pl.BlockSpec((tile, out_cols), lambda i: (i, 0)),
        compiler_params=pltpu.CompilerParams(
            dimension_semantics=("parallel",)),
    )(x0, x1, x2, x3, x4, w)

    return out


# speedup vs baseline: 2.3072x; 1.0346x over previous
import jax
import jax.numpy as jnp
from jax.experimental import pallas as pl
from jax.experimental.pallas import tpu as pltpu


def _residual_kernel(x0_ref, x1_ref, x2_ref, x3_ref, x4_ref, w_ref, out_ref):
    hf = w_ref.shape[1]
    x0 = x0_ref[...]
    w = w_ref[...]
    for h, x_ref in enumerate((x1_ref, x2_ref, x3_ref, x4_ref)):
        out_ref[:, h * hf:(h + 1) * hf] = jnp.dot(
            x_ref[...] - x0, w, preferred_element_type=jnp.float32
        ).astype(out_ref.dtype)


def _pick_tile(n, target):
    target = max(1, min(target, n))
    for t in range(target, 0, -1):
        if n % t == 0 and (t % 8 == 0 or t == n):
            return t
    return n


def kernel(x0, x1, x2, x3, x4, w, b):
    del b
    n, fin = x0.shape
    hf = w.shape[1]
    out_cols = 4 * hf

    tile = _pick_tile(n, 2048)
    n_tiles = n // tile

    x_spec = pl.BlockSpec((tile, fin), lambda i: (i, 0))
    out = pl.pallas_call(
        _residual_kernel,
        out_shape=jax.ShapeDtypeStruct((n, out_cols), x0.dtype),
        grid=(n_tiles,),
        in_specs=[x_spec] * 5 + [pl.BlockSpec((fin, hf), lambda i: (0, 0))],
        out_specs=pl.BlockSpec((tile, out_cols), lambda i: (i, 0)),
        compiler_params=pltpu.CompilerParams(
            dimension_semantics=("parallel",)),
    )(x0, x1, x2, x3, x4, w)

    return out
